# baseline (speedup 1.0000x reference)
import functools

import jax
import jax.numpy as jnp
from jax import lax
from jax.experimental import pallas as pl
from jax.experimental.pallas import tpu as pltpu
from jax.experimental.pallas import tpu_sc as plsc

_N = 10000
_NP = 10240
_E = 320000
_D = 128
_NC = 2
_NS = 16
_NTILES = _NC * _NS
_PER_TILE = 10240
_EPAD = _NTILES * _PER_TILE
_PT0 = 20480
_PT1 = 2 * _PER_TILE - _PT0
_GRP = 128
_SUPER = 1024
_NGRP = _SUPER // _GRP
_NSUPER = _PER_TILE // _SUPER
_RPT = _NP // _NS

_sc_mesh = plsc.VectorSubcoreMesh(core_axis_name="c", subcore_axis_name="s")


@functools.partial(
    pl.kernel,
    mesh=_sc_mesh,
    out_type=jax.ShapeDtypeStruct((_NC, _NP, _D), jnp.float32),
    scratch_types=[
        pltpu.VMEM((8, 128), jnp.int32),
        pltpu.VMEM((_GRP, _D), jnp.float32),
        pltpu.VMEM_SHARED((_NP, _D), jnp.float32),
        pltpu.SemaphoreType.DMA,
    ],
)
def _deg_call(ones_nodes, ones_blk, dst_hbm, d_hbm, dst_i, ones_v, deg_sh,
              sem):
    cid = lax.axis_index("c")
    sid = lax.axis_index("s")
    wid = cid * _NS + sid
    r0 = sid * _RPT
    pltpu.sync_copy(ones_nodes.at[pl.ds(r0, _RPT)], deg_sh.at[pl.ds(r0, _RPT)])
    pltpu.sync_copy(ones_blk, ones_v)
    plsc.subcore_barrier()

    row_base = wid * (_PER_TILE // 128)

    def super_body(i, carry):
        pltpu.sync_copy(dst_hbm.at[pl.ds(row_base + i * 8, 8)], dst_i)
        for j in range(_NGRP):
            pltpu.sync_copy(ones_v, deg_sh.at[dst_i.at[j]], add=True)
        return carry

    lax.fori_loop(0, _NSUPER, super_body, 0)
    plsc.subcore_barrier()
    pltpu.sync_copy(deg_sh.at[pl.ds(r0, _RPT)], d_hbm.at[cid, pl.ds(r0, _RPT)])


@functools.partial(
    pl.kernel,
    mesh=_sc_mesh,
    out_type=jax.ShapeDtypeStruct((_NC, _NP, _D), jnp.float32),
    scratch_types=[
        pltpu.VMEM((8, 128), jnp.int32),
        pltpu.VMEM((8, 128), jnp.int32),
        pltpu.VMEM((_GRP, _D), jnp.float32),
        pltpu.VMEM((_GRP, _D), jnp.float32),
        pltpu.VMEM_SHARED((_NP, _D), jnp.float32),
        pltpu.SemaphoreType.DMA,
        pltpu.SemaphoreType.DMA,
    ],
)
def _prop_call(y_hbm, src_hbm, dst_hbm, z_hbm, src_i, dst_i, rows_v0, rows_v1,
               acc_sh, sem0, sem1):
    cid = lax.axis_index("c")
    sid = lax.axis_index("s")
    r0 = sid * _RPT
    pltpu.sync_copy(y_hbm.at[pl.ds(r0, _RPT)], acc_sh.at[pl.ds(r0, _RPT)])
    plsc.subcore_barrier()

    per_tile = _PT0 + cid * (_PT1 - _PT0)
    sbase = cid * (_NS * _PT0 // _SUPER) + sid * (per_tile // _SUPER)
    nsuper = per_tile // _SUPER
    bufs = (rows_v0, rows_v1)
    sems = (sem0, sem1)

    def super_body(i, carry):
        pltpu.sync_copy(src_hbm.at[pl.ds((sbase + i) * 8, 8)], src_i)
        pltpu.sync_copy(dst_hbm.at[pl.ds((sbase + i) * 8, 8)], dst_i)
        handles = [pltpu.async_copy(y_hbm.at[src_i.at[0]], bufs[0], sems[0])]
        for j in range(_NGRP):
            if j + 1 < _NGRP:
                handles.append(
                    pltpu.async_copy(y_hbm.at[src_i.at[j + 1]],
                                     bufs[(j + 1) % 2], sems[(j + 1) % 2]))
            handles[j].wait()
            pltpu.sync_copy(bufs[j % 2], acc_sh.at[dst_i.at[j]], add=True)
        return carry

    lax.fori_loop(0, nsuper, super_body, 0)
    plsc.subcore_barrier()
    pltpu.sync_copy(acc_sh.at[pl.ds(r0, _RPT)], z_hbm.at[cid, pl.ds(r0, _RPT)])


_BLK = 2048


def _dinv_of(d0, d1):
    return lax.rsqrt(jnp.maximum(d0[:, 0:1] + d1[:, 0:1] - 1.0, 1.0))


def _tc_first_body(x_ref, w_ref, d0_ref, d1_ref, y_ref):
    dinv = _dinv_of(d0_ref[0], d1_ref[0])
    y_ref[...] = jnp.dot(x_ref[...], w_ref[...],
                         preferred_element_type=jnp.float32) * dinv


def _tc_mid_body(z0_ref, z1_ref, y_ref, d0_ref, d1_ref, b_ref, w_ref, o_ref):
    dinv = _dinv_of(d0_ref[0], d1_ref[0])
    z = z0_ref[0] + z1_ref[0] - y_ref[...]
    u = jnp.maximum(z * dinv + b_ref[...], 0.0)
    o_ref[...] = jnp.dot(u, w_ref[...],
                         preferred_element_type=jnp.float32) * dinv


def _tc_last_body(z0_ref, z1_ref, y_ref, d0_ref, d1_ref, b_ref, o_ref):
    dinv = _dinv_of(d0_ref[0], d1_ref[0])
    o_ref[...] = (z0_ref[0] + z1_ref[0] - y_ref[...]) * dinv + b_ref[...]


def _rows_spec():
    return pl.BlockSpec((_BLK, _D), lambda i: (i, 0))


def _z_spec(c):
    return pl.BlockSpec((1, _BLK, _D), lambda i: (c, i, 0))


def _deg_spec(c):
    return pl.BlockSpec((1, _BLK, _D), lambda i: (c, i, 0))


def _full_spec(r):
    return pl.BlockSpec((r, _D), lambda i: (0, 0))


def _tc_first(x, W, d):
    return pl.pallas_call(
        _tc_first_body,
        grid=(_NP // _BLK,),
        in_specs=[_rows_spec(), _full_spec(_D), _deg_spec(0), _deg_spec(1)],
        out_specs=_rows_spec(),
        out_shape=jax.ShapeDtypeStruct((_NP, _D), jnp.float32),
    )(x, W, d, d)


def _tc_mid(z, y, d, b, W):
    return pl.pallas_call(
        _tc_mid_body,
        grid=(_NP // _BLK,),
        in_specs=[_z_spec(0), _z_spec(1), _rows_spec(), _deg_spec(0),
                  _deg_spec(1), _full_spec(1), _full_spec(_D)],
        out_specs=_rows_spec(),
        out_shape=jax.ShapeDtypeStruct((_NP, _D), jnp.float32),
    )(z, z, y, d, d, b, W)


def _tc_last(z, y, d, b):
    return pl.pallas_call(
        _tc_last_body,
        grid=(_NP // _BLK,),
        in_specs=[_z_spec(0), _z_spec(1), _rows_spec(), _deg_spec(0),
                  _deg_spec(1), _full_spec(1)],
        out_specs=_rows_spec(),
        out_shape=jax.ShapeDtypeStruct((_NP, _D), jnp.float32),
    )(z, z, y, d, d, b)


def kernel(x, edge_index, W0, b0, W1, b1, W2, b2, W3, b3):
    src = edge_index[0].astype(jnp.int32)
    dst = edge_index[1].astype(jnp.int32)
    pad = _EPAD - _E
    src_p = jnp.concatenate([src, jnp.zeros((pad,), jnp.int32)])
    dst_p = jnp.concatenate([dst, jnp.full((pad,), _N, jnp.int32)])
    src_p = src_p.reshape(_EPAD // 128, 128)
    dst_p = dst_p.reshape(_EPAD // 128, 128)

    ones_nodes = jnp.ones((_NP, _D), jnp.float32)
    ones_blk = jnp.ones((_GRP, _D), jnp.float32)
    x = jnp.concatenate([x, jnp.zeros((_NP - _N, _D), jnp.float32)])

    d = _deg_call(ones_nodes, ones_blk, dst_p)

    y = _tc_first(x, W0, d)
    for W, b in ((W1, b0), (W2, b1), (W3, b2)):
        z = _prop_call(y, src_p, dst_p)
        y = _tc_mid(z, y, d, b.reshape(1, _D), W)
    z = _prop_call(y, src_p, dst_p)
    return _tc_last(z, y, d, b3.reshape(1, _D))[:_N]


# speedup vs baseline: 3.5132x; 3.5132x over previous
import functools

import jax
import jax.numpy as jnp
from jax import lax
from jax.experimental import pallas as pl
from jax.experimental.pallas import tpu as pltpu
from jax.experimental.pallas import tpu_sc as plsc

_N = 10000
_NP = 10240
_E = 320000
_D = 128
_NC = 2
_NS = 16
_NTILES = _NC * _NS
_PER_TILE = 10240
_EPAD = _NTILES * _PER_TILE
_PT0 = _PER_TILE
_PT1 = 2 * _PER_TILE - _PT0
_GRP = 128
_SUPER = 1024
_NGRP = _SUPER // _GRP
_NSUPER = _PER_TILE // _SUPER
_RPT = _NP // _NS

_sc_mesh = plsc.VectorSubcoreMesh(core_axis_name="c", subcore_axis_name="s")


@functools.partial(
    pl.kernel,
    mesh=_sc_mesh,
    out_type=jax.ShapeDtypeStruct((_NC, _NP, _D), jnp.float32),
    scratch_types=[
        pltpu.VMEM((8, 128), jnp.int32),
        pltpu.VMEM((_GRP, _D), jnp.float32),
        pltpu.VMEM_SHARED((_NP, _D), jnp.float32),
        pltpu.SemaphoreType.DMA,
    ],
)
def _deg_call(ones_nodes, ones_blk, dst_hbm, d_hbm, dst_i, ones_v, deg_sh,
              sem):
    cid = lax.axis_index("c")
    sid = lax.axis_index("s")
    wid = cid * _NS + sid
    r0 = sid * _RPT
    pltpu.sync_copy(ones_nodes.at[pl.ds(r0, _RPT)], deg_sh.at[pl.ds(r0, _RPT)])
    pltpu.sync_copy(ones_blk, ones_v)
    plsc.subcore_barrier()

    row_base = wid * (_PER_TILE // 128)

    def super_body(i, carry):
        pltpu.sync_copy(dst_hbm.at[pl.ds(row_base + i * 8, 8)], dst_i)
        for j in range(_NGRP):
            pltpu.sync_copy(ones_v, deg_sh.at[dst_i.at[j]], add=True)
        return carry

    lax.fori_loop(0, _NSUPER, super_body, 0)
    plsc.subcore_barrier()
    pltpu.sync_copy(deg_sh.at[pl.ds(r0, _RPT)], d_hbm.at[cid, pl.ds(r0, _RPT)])


@functools.partial(
    pl.kernel,
    mesh=_sc_mesh,
    out_type=jax.ShapeDtypeStruct((_NC, _NP, _D), jnp.float32),
    scratch_types=[
        pltpu.VMEM((8, 128), jnp.int32),
        pltpu.VMEM((8, 128), jnp.int32),
        pltpu.VMEM((_GRP, _D), jnp.float32),
        pltpu.VMEM((_GRP, _D), jnp.float32),
        pltpu.VMEM_SHARED((_NP, _D), jnp.float32),
        pltpu.SemaphoreType.DMA,
        pltpu.SemaphoreType.DMA,
    ],
)
def _prop_call(y_hbm, src_hbm, dst_hbm, z_hbm, src_i, dst_i, rows_v0, rows_v1,
               acc_sh, sem0, sem1):
    cid = lax.axis_index("c")
    sid = lax.axis_index("s")
    r0 = sid * _RPT
    pltpu.sync_copy(y_hbm.at[pl.ds(r0, _RPT)], acc_sh.at[pl.ds(r0, _RPT)])
    plsc.subcore_barrier()

    per_tile = _PT0 + cid * (_PT1 - _PT0)
    sbase = cid * (_NS * _PT0 // _SUPER) + sid * (per_tile // _SUPER)
    nsuper = per_tile // _SUPER
    bufs = (rows_v0, rows_v1)
    sems = (sem0, sem1)

    def super_body(i, carry):
        pltpu.sync_copy(src_hbm.at[pl.ds((sbase + i) * 8, 8)], src_i)
        pltpu.sync_copy(dst_hbm.at[pl.ds((sbase + i) * 8, 8)], dst_i)
        handles = [pltpu.async_copy(y_hbm.at[src_i.at[0]], bufs[0], sems[0])]
        for j in range(_NGRP):
            if j + 1 < _NGRP:
                handles.append(
                    pltpu.async_copy(y_hbm.at[src_i.at[j + 1]],
                                     bufs[(j + 1) % 2], sems[(j + 1) % 2]))
            handles[j].wait()
            pltpu.sync_copy(bufs[j % 2], acc_sh.at[dst_i.at[j]], add=True)
        return carry

    lax.fori_loop(0, nsuper, super_body, 0)
    plsc.subcore_barrier()
    pltpu.sync_copy(acc_sh.at[pl.ds(r0, _RPT)], z_hbm.at[cid, pl.ds(r0, _RPT)])


_BLK = 2048


def _dinv_of(d0, d1):
    return lax.rsqrt(jnp.maximum(d0[:, 0:1] + d1[:, 0:1] - 1.0, 1.0))


def _tc_first_body(x_ref, w_ref, d0_ref, d1_ref, y_ref):
    dinv = _dinv_of(d0_ref[0], d1_ref[0])
    y_ref[...] = jnp.dot(x_ref[...], w_ref[...],
                         preferred_element_type=jnp.float32) * dinv


def _tc_mid_body(z0_ref, z1_ref, y_ref, d0_ref, d1_ref, b_ref, w_ref, o_ref):
    dinv = _dinv_of(d0_ref[0], d1_ref[0])
    z = z0_ref[0] + z1_ref[0] - y_ref[...]
    u = jnp.maximum(z * dinv + b_ref[...], 0.0)
    o_ref[...] = jnp.dot(u, w_ref[...],
                         preferred_element_type=jnp.float32) * dinv


def _tc_last_body(z0_ref, z1_ref, y_ref, d0_ref, d1_ref, b_ref, o_ref):
    dinv = _dinv_of(d0_ref[0], d1_ref[0])
    o_ref[...] = (z0_ref[0] + z1_ref[0] - y_ref[...]) * dinv + b_ref[...]


def _rows_spec():
    return pl.BlockSpec((_BLK, _D), lambda i: (i, 0))


def _z_spec(c):
    return pl.BlockSpec((1, _BLK, _D), lambda i: (c, i, 0))


def _deg_spec(c):
    return pl.BlockSpec((1, _BLK, _D), lambda i: (c, i, 0))


def _full_spec(r):
    return pl.BlockSpec((r, _D), lambda i: (0, 0))


def _tc_first(x, W, d):
    return pl.pallas_call(
        _tc_first_body,
        grid=(_NP // _BLK,),
        in_specs=[_rows_spec(), _full_spec(_D), _deg_spec(0), _deg_spec(1)],
        out_specs=_rows_spec(),
        out_shape=jax.ShapeDtypeStruct((_NP, _D), jnp.float32),
    )(x, W, d, d)


def _tc_mid(z, y, d, b, W):
    return pl.pallas_call(
        _tc_mid_body,
        grid=(_NP // _BLK,),
        in_specs=[_z_spec(0), _z_spec(1), _rows_spec(), _deg_spec(0),
                  _deg_spec(1), _full_spec(1), _full_spec(_D)],
        out_specs=_rows_spec(),
        out_shape=jax.ShapeDtypeStruct((_NP, _D), jnp.float32),
    )(z, z, y, d, d, b, W)


def _tc_last(z, y, d, b):
    return pl.pallas_call(
        _tc_last_body,
        grid=(_NP // _BLK,),
        in_specs=[_z_spec(0), _z_spec(1), _rows_spec(), _deg_spec(0),
                  _deg_spec(1), _full_spec(1)],
        out_specs=_rows_spec(),
        out_shape=jax.ShapeDtypeStruct((_NP, _D), jnp.float32),
    )(z, z, y, d, d, b)


def kernel(x, edge_index, W0, b0, W1, b1, W2, b2, W3, b3):
    src = edge_index[0].astype(jnp.int32)
    dst = edge_index[1].astype(jnp.int32)
    pad = _EPAD - _E
    pad_dst = _N + (jnp.arange(pad, dtype=jnp.int32) % (_NP - _N))
    pad_src = jnp.arange(pad, dtype=jnp.int32) % _N
    src_p = jnp.concatenate([src, pad_src])
    dst_p = jnp.concatenate([dst, pad_dst])
    src_p = src_p.reshape(_EPAD // 128, 128)
    dst_p = dst_p.reshape(_EPAD // 128, 128)

    ones_nodes = jnp.ones((_NP, _D), jnp.float32)
    ones_blk = jnp.ones((_GRP, _D), jnp.float32)
    x = jnp.concatenate([x, jnp.zeros((_NP - _N, _D), jnp.float32)])

    d = _deg_call(ones_nodes, ones_blk, dst_p)

    y = _tc_first(x, W0, d)
    for W, b in ((W1, b0), (W2, b1), (W3, b2)):
        z = _prop_call(y, src_p, dst_p)
        y = _tc_mid(z, y, d, b.reshape(1, _D), W)
    z = _prop_call(y, src_p, dst_p)
    return _tc_last(z, y, d, b3.reshape(1, _D))[:_N]
